# baseline (speedup 1.0000x reference)
import functools

import jax
import jax.numpy as jnp
from jax import lax
from jax.experimental import pallas as pl
from jax.experimental.pallas import tpu as pltpu
from jax.experimental.pallas import tpu_sc as plsc

HEAD_DIM = 128

_NUM_CORES = 2
_NUM_SUBCORES = 16
_NUM_WORKERS = _NUM_CORES * _NUM_SUBCORES
_CHUNK = 64
_NSLOT = 3
_WSLOT = 2


@functools.lru_cache(maxsize=None)
def _make_gather(batch, seq):
    n_rows = batch * seq
    b_per_w = n_rows // _NUM_WORKERS
    w_per_b = seq // b_per_w
    n_chunks = b_per_w // _CHUNK
    mesh = plsc.VectorSubcoreMesh(core_axis_name="c", subcore_axis_name="s")

    buf_types = [pltpu.VMEM((_CHUNK, HEAD_DIM), jnp.float32)
                 for _ in range(2 * _NSLOT)]
    sem_types = [pltpu.SemaphoreType.DMA for _ in range(2 * _NSLOT)]

    @functools.partial(
        pl.kernel,
        mesh=mesh,
        out_type=[
            jax.ShapeDtypeStruct((batch, 1, seq, HEAD_DIM), jnp.float32),
            jax.ShapeDtypeStruct((batch, 1, seq, HEAD_DIM), jnp.float32),
        ],
        scratch_types=[pltpu.VMEM((b_per_w,), jnp.int32)]
                      + buf_types + sem_types
                      + [pltpu.VMEM_SHARED(
                             (_NUM_SUBCORES, _WSLOT, _CHUNK, HEAD_DIM),
                             jnp.float32) for _ in range(2)],
    )
    def gather_kernel(cos_hbm, sin_hbm, idx_hbm, cos_out, sin_out,
                      idx_v, *bufs_and_sems):
        cbufs = bufs_and_sems[0:_NSLOT]
        sbufs = bufs_and_sems[_NSLOT:2 * _NSLOT]
        gsems = bufs_and_sems[2 * _NSLOT:3 * _NSLOT]
        wsems = bufs_and_sems[3 * _NSLOT:4 * _NSLOT]
        cslab, sslab = bufs_and_sems[4 * _NSLOT:4 * _NSLOT + 2]

        sid = lax.axis_index("s")
        wid = sid * _NUM_CORES + lax.axis_index("c")
        bi = wid // w_per_b
        inner = (wid % w_per_b) * b_per_w
        pltpu.sync_copy(idx_hbm.at[bi, pl.ds(inner, b_per_w)], idx_v)

        def issue_gather(i):
            s = i % _NSLOT
            sl = idx_v.at[pl.ds(i * _CHUNK, _CHUNK)]
            return (pltpu.async_copy(cos_hbm.at[sl], cbufs[s], gsems[s]),
                    pltpu.async_copy(sin_hbm.at[sl], sbufs[s], gsems[s]))

        def stage(i):
            s = i % _NSLOT
            w = i % _WSLOT
            pltpu.sync_copy(cbufs[s], cslab.at[sid, w])
            pltpu.sync_copy(sbufs[s], sslab.at[sid, w])

        def issue_write(i):
            w = i % _WSLOT
            rows = pl.ds(inner + i * _CHUNK, _CHUNK)
            return (pltpu.async_copy(cslab.at[sid, w],
                                     cos_out.at[bi, 0, rows, :], wsems[w]),
                    pltpu.async_copy(sslab.at[sid, w],
                                     sin_out.at[bi, 0, rows, :], wsems[w]))

        gh = {}
        wh = {}
        for i in range(min(_NSLOT, n_chunks)):
            gh[i] = issue_gather(i)
        for i in range(n_chunks):
            for h in gh.pop(i):
                h.wait()
            if i >= _WSLOT:
                for h in wh.pop(i - _WSLOT):
                    h.wait()
            stage(i)
            if i + _NSLOT < n_chunks:
                gh[i + _NSLOT] = issue_gather(i + _NSLOT)
            wh[i] = issue_write(i)
        for i in range(max(0, n_chunks - _WSLOT), n_chunks):
            for h in wh.pop(i):
                h.wait()

    return gather_kernel


def kernel(cos_cached, sin_cached, position_ids, batch_size, seq_len):
    del batch_size, seq_len
    b, s = position_ids.shape
    cos_table = cos_cached[0, 0]
    sin_table = sin_cached[0, 0]
    cos, sin = _make_gather(b, s)(cos_table, sin_table, position_ids)
    return (cos, sin)


# speedup vs baseline: 1.0521x; 1.0521x over previous
import functools

import jax
import jax.numpy as jnp
from jax import lax
from jax.experimental import pallas as pl
from jax.experimental.pallas import tpu as pltpu
from jax.experimental.pallas import tpu_sc as plsc

HEAD_DIM = 128

_NUM_CORES = 2
_NUM_SUBCORES = 16
_NUM_WORKERS = _NUM_CORES * _NUM_SUBCORES
_CHUNK = 128
_NSLOT = 3


@functools.lru_cache(maxsize=None)
def _make_gather(batch, seq):
    n_rows = batch * seq
    b_per_w = n_rows // _NUM_WORKERS
    w_per_b = seq // b_per_w
    n_chunks = b_per_w // _CHUNK
    mesh = plsc.VectorSubcoreMesh(core_axis_name="c", subcore_axis_name="s")

    buf_types = [pltpu.VMEM((_CHUNK, HEAD_DIM), jnp.float32)
                 for _ in range(2 * _NSLOT)]
    sem_types = [pltpu.SemaphoreType.DMA for _ in range(2 * _NSLOT)]

    @functools.partial(
        pl.kernel,
        mesh=mesh,
        out_type=[
            jax.ShapeDtypeStruct((batch, 1, seq, HEAD_DIM), jnp.float32),
            jax.ShapeDtypeStruct((batch, 1, seq, HEAD_DIM), jnp.float32),
        ],
        scratch_types=[pltpu.VMEM((b_per_w,), jnp.int32)]
                      + buf_types + sem_types,
    )
    def gather_kernel(cos_hbm, sin_hbm, idx_hbm, cos_out, sin_out,
                      idx_v, *bufs_and_sems):
        cbufs = bufs_and_sems[0:_NSLOT]
        sbufs = bufs_and_sems[_NSLOT:2 * _NSLOT]
        gsems = bufs_and_sems[2 * _NSLOT:3 * _NSLOT]
        wsems = bufs_and_sems[3 * _NSLOT:4 * _NSLOT]

        wid = lax.axis_index("s") * _NUM_CORES + lax.axis_index("c")
        bi = wid // w_per_b
        inner = (wid % w_per_b) * b_per_w
        pltpu.sync_copy(idx_hbm.at[bi, pl.ds(inner, b_per_w)], idx_v)

        def issue_gather(i):
            s = i % _NSLOT
            sl = idx_v.at[pl.ds(i * _CHUNK, _CHUNK)]
            return (pltpu.async_copy(cos_hbm.at[sl], cbufs[s], gsems[s]),
                    pltpu.async_copy(sin_hbm.at[sl], sbufs[s], gsems[s]))

        def issue_write(i):
            s = i % _NSLOT
            rows = pl.ds(inner + i * _CHUNK, _CHUNK)
            return (pltpu.async_copy(cbufs[s], cos_out.at[bi, 0, rows, :],
                                     wsems[s]),
                    pltpu.async_copy(sbufs[s], sin_out.at[bi, 0, rows, :],
                                     wsems[s]))

        gh = {}
        wh = {}
        for i in range(min(2, n_chunks)):
            gh[i] = issue_gather(i)
        for i in range(n_chunks):
            if i >= 1:
                for h in wh.pop(i - 1):
                    h.wait()
            if i + 2 < n_chunks:
                gh[i + 2] = issue_gather(i + 2)
            for h in gh.pop(i):
                h.wait()
            wh[i] = issue_write(i)
        for h in wh.pop(n_chunks - 1):
            h.wait()

    return gather_kernel


def kernel(cos_cached, sin_cached, position_ids, batch_size, seq_len):
    del batch_size, seq_len
    b, s = position_ids.shape
    cos_table = cos_cached[0, 0]
    sin_table = sin_cached[0, 0]
    cos, sin = _make_gather(b, s)(cos_table, sin_table, position_ids)
    return (cos, sin)
